# baseline (speedup 1.0000x reference)
import functools

import jax
import jax.numpy as jnp
from jax import lax
from jax.experimental import pallas as pl
from jax.experimental.pallas import tpu as pltpu
from jax.experimental.pallas import tpu_sc as plsc

NSEG = 27
NPAIR = 20000
UNITS = 128
ROWS = 512
COLS = 1024

NCORE = 2
NSUB = 16
SEG_PER_CORE = 14
NCHUNK = 10
CHUNK = 125
NLANE = 16
NVEC = UNITS // NLANE
NBUF = 5
ROW_UNROLL = 5


def _sc_body(table_hbm, idx_hbm, out_hbm,
             idx_v, b0, b1, b2, b3, b4, acc_v, acc_sh,
             s0, s1, s2, s3, s4, si):
    c = lax.axis_index("c")
    s = lax.axis_index("s")
    bufs = (b0, b1, b2, b3, b4)
    sems = (s0, s1, s2, s3, s4)

    nseg = jnp.where(c == 0, SEG_PER_CORE, NSEG - SEG_PER_CORE)
    nq = nseg * NCHUNK

    zero16 = jnp.zeros((NLANE,), jnp.float32)

    def _zero(kk, carry):
        for u in range(NVEC):
            acc_v[kk, pl.ds(u * NLANE, NLANE)] = zero16
        return carry

    lax.fori_loop(0, NSUB, _zero, 0)

    @pl.when(s == 0)
    def _():
        pltpu.sync_copy(acc_v, acc_sh)

    plsc.subcore_barrier()

    def idx_start(kk, carry):
        pltpu.async_copy(idx_hbm.at[c * SEG_PER_CORE + kk, s],
                         idx_v.at[kk], si)
        return carry

    lax.fori_loop(0, nseg, idx_start, 0)

    def idx_wait(kk, carry):
        pltpu.make_async_copy(idx_hbm.at[0, 0], idx_v.at[kk], si).wait()
        return carry

    lax.fori_loop(0, nseg, idx_wait, 0)

    for b in range(NBUF):
        pltpu.async_copy(table_hbm.at[idx_v.at[0, b]], bufs[b], sems[b])

    def q_body(g, carry):
        for b in range(NBUF):
            q = g * NBUF + b
            buf, sem = bufs[b], sems[b]
            pltpu.make_async_copy(table_hbm.at[idx_v.at[0, 0]],
                                  buf, sem).wait()

            kk = (q * 6554) >> 16

            def row_body(i, a):
                out = a
                for r in range(ROW_UNROLL):
                    row = i * ROW_UNROLL + r
                    out = tuple(
                        out[u] + buf[row, pl.ds(u * NLANE, NLANE)]
                        for u in range(NVEC)
                    )
                return out

            acc = lax.fori_loop(0, CHUNK // ROW_UNROLL, row_body,
                                tuple(zero16 for _ in range(NVEC)))
            for u in range(NVEC):
                plsc.addupdate(acc_v.at[kk, pl.ds(u * NLANE, NLANE)], acc[u])

            qn = q + NBUF

            @pl.when(qn < nq)
            def _():
                kk2 = (qn * 6554) >> 16
                ch2 = qn - kk2 * NCHUNK
                pltpu.async_copy(table_hbm.at[idx_v.at[kk2, ch2]], buf, sem)
        return carry

    lax.fori_loop(0, nq // NBUF, q_body, 0)

    row_ids = lax.iota(jnp.int32, NLANE)
    pltpu.sync_copy(acc_v, acc_sh.at[row_ids], add=True)
    plsc.subcore_barrier()

    @pl.when(s == 0)
    def _():
        pltpu.sync_copy(acc_sh, acc_v)
        inv = jnp.full((NLANE,), 1.0 / NPAIR, jnp.float32)

        def scale_body(kk, carry):
            for u in range(NVEC):
                sl = pl.ds(u * NLANE, NLANE)
                acc_v[kk, sl] = acc_v[kk, sl] * inv
            return carry

        lax.fori_loop(0, NSUB, scale_body, 0)
        pltpu.sync_copy(acc_v, out_hbm.at[c])


@jax.jit
def _node_pool_sc(table, idx4):
    mesh = plsc.VectorSubcoreMesh(core_axis_name="c", subcore_axis_name="s")
    k = functools.partial(
        pl.kernel,
        out_type=jax.ShapeDtypeStruct((NCORE, NSUB, UNITS), jnp.float32),
        mesh=mesh,
        scratch_types=[
            pltpu.VMEM((SEG_PER_CORE, NCHUNK, CHUNK), jnp.int32),
            pltpu.VMEM((CHUNK, UNITS), jnp.float32),
            pltpu.VMEM((CHUNK, UNITS), jnp.float32),
            pltpu.VMEM((CHUNK, UNITS), jnp.float32),
            pltpu.VMEM((CHUNK, UNITS), jnp.float32),
            pltpu.VMEM((CHUNK, UNITS), jnp.float32),
            pltpu.VMEM((NSUB, UNITS), jnp.float32),
            pltpu.VMEM_SHARED((NSUB, UNITS), jnp.float32),
            pltpu.SemaphoreType.DMA,
            pltpu.SemaphoreType.DMA,
            pltpu.SemaphoreType.DMA,
            pltpu.SemaphoreType.DMA,
            pltpu.SemaphoreType.DMA,
            pltpu.SemaphoreType.DMA,
        ],
    )(_sc_body)
    return k(table, idx4)


def kernel(inputs, pairs):
    table = inputs.reshape(ROWS * COLS, UNITS)
    p4 = pairs.reshape(NSEG, NSUB, NCHUNK, CHUNK, 2)
    w = jnp.array([COLS, 1], jnp.float32)
    idx = jnp.einsum('ksclt,t->kscl', p4.astype(jnp.float32), w,
                     preferred_element_type=jnp.float32).astype(jnp.int32)
    out = _node_pool_sc(table, idx)
    return jnp.concatenate(
        [out[0, :SEG_PER_CORE], out[1, :NSEG - SEG_PER_CORE]], axis=0)


# speedup vs baseline: 1.0672x; 1.0672x over previous
import functools

import jax
import jax.numpy as jnp
from jax import lax
from jax.experimental import pallas as pl
from jax.experimental.pallas import tpu as pltpu
from jax.experimental.pallas import tpu_sc as plsc

NSEG = 27
NPAIR = 20000
UNITS = 128
ROWS = 512
COLS = 1024

NCORE = 2
NSUB = 16
NFULL = 13
KSHARED = 13
SROW = 13
NCHUNK = 10
NHALF = 5
CHUNK = 125
NLANE = 16
NVEC = UNITS // NLANE
NBUF = 5
ROW_UNROLL = 5


def _sc_body(table_hbm, idx_hbm, out_hbm,
             idx_v, sidx_v, b0, b1, b2, b3, b4, acc_v, acc_sh,
             s0, s1, s2, s3, s4, si):
    c = lax.axis_index("c")
    s = lax.axis_index("s")
    bufs = (b0, b1, b2, b3, b4)
    sems = (s0, s1, s2, s3, s4)

    zero16 = jnp.zeros((NLANE,), jnp.float32)

    def _zero(kk, carry):
        for u in range(NVEC):
            acc_v[kk, pl.ds(u * NLANE, NLANE)] = zero16
        return carry

    lax.fori_loop(0, NSUB, _zero, 0)

    @pl.when(s == 0)
    def _():
        pltpu.sync_copy(acc_v, acc_sh)

    plsc.subcore_barrier()

    def idx_start(kk, carry):
        pltpu.async_copy(idx_hbm.at[c * (NFULL + 1) + kk, pl.ds(2 * s, 2)],
                         idx_v.at[kk], si)
        return carry

    lax.fori_loop(0, NFULL, idx_start, 0)
    pltpu.async_copy(idx_hbm.at[KSHARED, c * NSUB + s], sidx_v, si)

    def idx_wait(kk, carry):
        pltpu.make_async_copy(idx_hbm.at[0, pl.ds(0, 2)], idx_v.at[kk],
                              si).wait()
        return carry

    lax.fori_loop(0, NFULL, idx_wait, 0)
    pltpu.make_async_copy(idx_hbm.at[0, 0], sidx_v, si).wait()

    def accumulate(buf):
        def row_body(i, a):
            out = a
            for r in range(ROW_UNROLL):
                row = i * ROW_UNROLL + r
                out = tuple(
                    out[u] + buf[row, pl.ds(u * NLANE, NLANE)]
                    for u in range(NVEC)
                )
            return out

        return lax.fori_loop(0, CHUNK // ROW_UNROLL, row_body,
                             tuple(zero16 for _ in range(NVEC)))

    for b in range(NBUF):
        pltpu.async_copy(table_hbm.at[idx_v.at[0, 0, b]], bufs[b], sems[b])

    def seg_body(kk, carry):
        for ch in range(NCHUNK):
            buf, sem = bufs[ch % NBUF], sems[ch % NBUF]
            pltpu.make_async_copy(table_hbm.at[idx_v.at[0, 0, 0]],
                                  buf, sem).wait()
            acc = accumulate(buf)
            for u in range(NVEC):
                plsc.addupdate(acc_v.at[kk, pl.ds(u * NLANE, NLANE)], acc[u])

            if ch < NBUF:
                pltpu.async_copy(table_hbm.at[idx_v.at[kk, 1, ch]], buf, sem)
            else:
                @pl.when(kk + 1 < NFULL)
                def _():
                    pltpu.async_copy(
                        table_hbm.at[idx_v.at[kk + 1, 0, ch - NBUF]],
                        buf, sem)

                @pl.when(kk + 1 == NFULL)
                def _():
                    pltpu.async_copy(table_hbm.at[sidx_v.at[ch - NBUF]],
                                     buf, sem)
        return carry

    lax.fori_loop(0, NFULL, seg_body, 0)

    for ch in range(NHALF):
        buf, sem = bufs[ch], sems[ch]
        pltpu.make_async_copy(table_hbm.at[idx_v.at[0, 0, 0]],
                              buf, sem).wait()
        acc = accumulate(buf)
        for u in range(NVEC):
            plsc.addupdate(acc_v.at[SROW, pl.ds(u * NLANE, NLANE)], acc[u])

    row_ids = lax.iota(jnp.int32, NLANE)
    pltpu.sync_copy(acc_v, acc_sh.at[row_ids], add=True)
    plsc.subcore_barrier()

    @pl.when(s == 0)
    def _():
        pltpu.sync_copy(acc_sh, acc_v)
        inv = jnp.full((NLANE,), 1.0 / NPAIR, jnp.float32)

        def scale_body(kk, carry):
            for u in range(NVEC):
                sl = pl.ds(u * NLANE, NLANE)
                acc_v[kk, sl] = acc_v[kk, sl] * inv
            return carry

        lax.fori_loop(0, NSUB, scale_body, 0)
        pltpu.sync_copy(acc_v, out_hbm.at[c])


@jax.jit
def _node_pool_sc(table, idx4):
    mesh = plsc.VectorSubcoreMesh(core_axis_name="c", subcore_axis_name="s")
    k = functools.partial(
        pl.kernel,
        out_type=jax.ShapeDtypeStruct((NCORE, NSUB, UNITS), jnp.float32),
        mesh=mesh,
        scratch_types=[
            pltpu.VMEM((NFULL, 2, NHALF, CHUNK), jnp.int32),
            pltpu.VMEM((NHALF, CHUNK), jnp.int32),
            pltpu.VMEM((CHUNK, UNITS), jnp.float32),
            pltpu.VMEM((CHUNK, UNITS), jnp.float32),
            pltpu.VMEM((CHUNK, UNITS), jnp.float32),
            pltpu.VMEM((CHUNK, UNITS), jnp.float32),
            pltpu.VMEM((CHUNK, UNITS), jnp.float32),
            pltpu.VMEM((NSUB, UNITS), jnp.float32),
            pltpu.VMEM_SHARED((NSUB, UNITS), jnp.float32),
            pltpu.SemaphoreType.DMA,
            pltpu.SemaphoreType.DMA,
            pltpu.SemaphoreType.DMA,
            pltpu.SemaphoreType.DMA,
            pltpu.SemaphoreType.DMA,
            pltpu.SemaphoreType.DMA,
        ],
    )(_sc_body)
    return k(table, idx4)


def kernel(inputs, pairs):
    table = inputs.reshape(ROWS * COLS, UNITS)
    p4 = pairs.reshape(NSEG, 2 * NSUB, NHALF, CHUNK, 2)
    idx = p4[..., 0] * COLS + p4[..., 1]
    out = _node_pool_sc(table, idx)
    shared = (out[0, SROW] + out[1, SROW])[None, :]
    return jnp.concatenate(
        [out[0, :NFULL], shared, out[1, :NFULL]], axis=0)
